# baseline (speedup 1.0000x reference)
import functools

import jax
import jax.numpy as jnp
from jax.experimental import pallas as pl
from jax.experimental.pallas import tpu as pltpu

B, C, H, W = 16, 512, 32, 32
HW = H * W
EMBED_DIM, TEXT_DIM, NUM_TOPK = 256, 768, 100


def _density_body(feat_ref, wd_ref, bd_ref, dens_ref):
    f = feat_ref[0]
    wd = wd_ref[...]
    lg = jnp.dot(wd, f, preferred_element_type=jnp.float32) + bd_ref[...]
    dens_ref[0] = jax.nn.sigmoid(lg)


def _mask_body(dens_ref, mask_ref):
    d = dens_ref[...]
    keys = jax.lax.bitcast_convert_type(d, jnp.int32)
    k_count = jnp.int32(NUM_TOPK)

    def val_step(_, carry):
        lo, hi = carry
        mid = lo + (hi - lo) // 2
        cnt = jnp.sum((keys > mid).astype(jnp.int32), axis=1, keepdims=True)
        pred = cnt < k_count
        return jnp.where(pred, lo, mid), jnp.where(pred, mid, hi)

    lo0 = jnp.full((B, 1), -1, jnp.int32)
    hi0 = jnp.full((B, 1), jnp.iinfo(jnp.int32).max - 1, jnp.int32)
    _, t_val = jax.lax.fori_loop(0, 32, val_step, (lo0, hi0))

    n_gt = jnp.sum((keys > t_val).astype(jnp.int32), axis=1, keepdims=True)
    need = k_count - n_gt
    eq = keys == t_val
    idx = jax.lax.broadcasted_iota(jnp.int32, (B, HW), 1)

    def idx_step(_, carry):
        lo, hi = carry
        mid = lo + (hi - lo) // 2
        cnt = jnp.sum((eq & (idx < mid)).astype(jnp.int32), axis=1, keepdims=True)
        pred = cnt >= need
        return jnp.where(pred, lo, mid), jnp.where(pred, mid, hi)

    lo0 = jnp.zeros((B, 1), jnp.int32)
    hi0 = jnp.full((B, 1), HW, jnp.int32)
    _, x_star = jax.lax.fori_loop(0, 11, idx_step, (lo0, hi0))

    mask = (keys > t_val) | (eq & (idx < x_star))
    mask_ref[...] = mask.astype(jnp.float32)


def _fusion_body(feat_ref, mask_ref, wsp_ref, wout_ref, wtext_ref, temb_ref,
                 bsp_ref, btext_ref, bout_ref, out_ref):
    f = feat_ref[0]
    m = mask_ref[0]
    tcol = (jnp.dot(wtext_ref[...], temb_ref[...],
                    preferred_element_type=jnp.float32)
            + btext_ref[...] + bsp_ref[...])
    z1 = jnp.dot(wsp_ref[...], f, preferred_element_type=jnp.float32) + tcol
    z2 = jnp.dot(wout_ref[...], z1, preferred_element_type=jnp.float32) + bout_ref[...]
    out_ref[0] = jnp.where(m > 0.0, z2, f)


@functools.partial(jax.jit, static_argnames=())
def kernel(feat, text_emb, Wd, bd, W_sp, b_sp, W_text, b_text, W_out, b_out):
    b, c, h, w = feat.shape
    feat3 = feat.reshape(b, c, h * w)

    density = pl.pallas_call(
        _density_body,
        grid=(b,),
        in_specs=[
            pl.BlockSpec((1, c, h * w), lambda i: (i, 0, 0)),
            pl.BlockSpec((1, c), lambda i: (0, 0)),
            pl.BlockSpec((1, 1), lambda i: (0, 0)),
        ],
        out_specs=pl.BlockSpec((1, 1, h * w), lambda i: (i, 0, 0)),
        out_shape=jax.ShapeDtypeStruct((b, 1, h * w), jnp.float32),
        compiler_params=pltpu.CompilerParams(
            dimension_semantics=("arbitrary",)),
    )(feat3, Wd.reshape(1, c), bd.reshape(1, 1))

    mask = pl.pallas_call(
        _mask_body,
        out_shape=jax.ShapeDtypeStruct((b, h * w), jnp.float32),
    )(density.reshape(b, h * w))

    out = pl.pallas_call(
        _fusion_body,
        grid=(b,),
        in_specs=[
            pl.BlockSpec((1, c, h * w), lambda i: (i, 0, 0)),
            pl.BlockSpec((1, 1, h * w), lambda i: (i, 0, 0)),
            pl.BlockSpec((EMBED_DIM, c), lambda i: (0, 0)),
            pl.BlockSpec((c, EMBED_DIM), lambda i: (0, 0)),
            pl.BlockSpec((EMBED_DIM, TEXT_DIM), lambda i: (0, 0)),
            pl.BlockSpec((TEXT_DIM, 1), lambda i: (0, 0)),
            pl.BlockSpec((EMBED_DIM, 1), lambda i: (0, 0)),
            pl.BlockSpec((EMBED_DIM, 1), lambda i: (0, 0)),
            pl.BlockSpec((c, 1), lambda i: (0, 0)),
        ],
        out_specs=pl.BlockSpec((1, c, h * w), lambda i: (i, 0, 0)),
        out_shape=jax.ShapeDtypeStruct((b, c, h * w), jnp.float32),
        compiler_params=pltpu.CompilerParams(
            dimension_semantics=("arbitrary",)),
    )(feat3, mask.reshape(b, 1, h * w), W_sp, W_out, W_text,
      text_emb.reshape(TEXT_DIM, 1), b_sp.reshape(EMBED_DIM, 1),
      b_text.reshape(EMBED_DIM, 1), b_out.reshape(c, 1))

    return out.reshape(b, c, h, w)


# speedup vs baseline: 1.0025x; 1.0025x over previous
import functools

import jax
import jax.numpy as jnp
from jax.experimental import pallas as pl
from jax.experimental.pallas import tpu as pltpu

B, C, H, W = 16, 512, 32, 32
HW = H * W
EMBED_DIM, TEXT_DIM, NUM_TOPK = 256, 768, 100


def _density_body(feat_ref, wd_ref, bd_ref, dens_ref):
    f = feat_ref[0]
    wd = wd_ref[...]
    lg = jnp.dot(wd, f, preferred_element_type=jnp.float32) + bd_ref[...]
    dens_ref[0] = jax.nn.sigmoid(lg)


def _mask_body(dens_ref, mask_ref):
    d = dens_ref[...]
    keys = jax.lax.bitcast_convert_type(d, jnp.int32)
    k_count = jnp.int32(NUM_TOPK)

    def val_step(_, carry):
        lo, hi = carry
        mid = lo + (hi - lo) // 2
        cnt = jnp.sum((keys > mid).astype(jnp.int32), axis=1, keepdims=True)
        pred = cnt < k_count
        return jnp.where(pred, lo, mid), jnp.where(pred, mid, hi)

    lo0 = jnp.full((B, 1), -1, jnp.int32)
    hi0 = jnp.full((B, 1), jnp.iinfo(jnp.int32).max - 1, jnp.int32)
    _, t_val = jax.lax.fori_loop(0, 32, val_step, (lo0, hi0))

    n_gt = jnp.sum((keys > t_val).astype(jnp.int32), axis=1, keepdims=True)
    need = k_count - n_gt
    eq = keys == t_val
    idx = jax.lax.broadcasted_iota(jnp.int32, (B, HW), 1)

    def idx_step(_, carry):
        lo, hi = carry
        mid = lo + (hi - lo) // 2
        cnt = jnp.sum((eq & (idx < mid)).astype(jnp.int32), axis=1, keepdims=True)
        pred = cnt >= need
        return jnp.where(pred, lo, mid), jnp.where(pred, mid, hi)

    lo0 = jnp.zeros((B, 1), jnp.int32)
    hi0 = jnp.full((B, 1), HW, jnp.int32)
    _, x_star = jax.lax.fori_loop(0, 11, idx_step, (lo0, hi0))

    mask = (keys > t_val) | (eq & (idx < x_star))
    mask_ref[...] = mask.astype(jnp.float32)


def _fusion_body(feat_ref, mask_ref, wsp_ref, wout_ref, wtext_ref, temb_ref,
                 bsp_ref, btext_ref, bout_ref, out_ref):
    f = feat_ref[0]
    m = mask_ref[0]
    tcol = (jnp.dot(wtext_ref[...], temb_ref[...],
                    preferred_element_type=jnp.float32)
            + btext_ref[...] + bsp_ref[...])
    z1 = jnp.dot(wsp_ref[...].astype(jnp.bfloat16), f.astype(jnp.bfloat16),
                 preferred_element_type=jnp.float32) + tcol
    z2 = jnp.dot(wout_ref[...].astype(jnp.bfloat16), z1.astype(jnp.bfloat16),
                 preferred_element_type=jnp.float32) + bout_ref[...]
    out_ref[0] = jnp.where(m > 0.0, z2, f)


@functools.partial(jax.jit, static_argnames=())
def kernel(feat, text_emb, Wd, bd, W_sp, b_sp, W_text, b_text, W_out, b_out):
    b, c, h, w = feat.shape
    feat3 = feat.reshape(b, c, h * w)

    density = pl.pallas_call(
        _density_body,
        grid=(b,),
        in_specs=[
            pl.BlockSpec((1, c, h * w), lambda i: (i, 0, 0)),
            pl.BlockSpec((1, c), lambda i: (0, 0)),
            pl.BlockSpec((1, 1), lambda i: (0, 0)),
        ],
        out_specs=pl.BlockSpec((1, 1, h * w), lambda i: (i, 0, 0)),
        out_shape=jax.ShapeDtypeStruct((b, 1, h * w), jnp.float32),
        compiler_params=pltpu.CompilerParams(
            dimension_semantics=("arbitrary",)),
    )(feat3, Wd.reshape(1, c), bd.reshape(1, 1))

    mask = pl.pallas_call(
        _mask_body,
        out_shape=jax.ShapeDtypeStruct((b, h * w), jnp.float32),
    )(density.reshape(b, h * w))

    out = pl.pallas_call(
        _fusion_body,
        grid=(b,),
        in_specs=[
            pl.BlockSpec((1, c, h * w), lambda i: (i, 0, 0)),
            pl.BlockSpec((1, 1, h * w), lambda i: (i, 0, 0)),
            pl.BlockSpec((EMBED_DIM, c), lambda i: (0, 0)),
            pl.BlockSpec((c, EMBED_DIM), lambda i: (0, 0)),
            pl.BlockSpec((EMBED_DIM, TEXT_DIM), lambda i: (0, 0)),
            pl.BlockSpec((TEXT_DIM, 1), lambda i: (0, 0)),
            pl.BlockSpec((EMBED_DIM, 1), lambda i: (0, 0)),
            pl.BlockSpec((EMBED_DIM, 1), lambda i: (0, 0)),
            pl.BlockSpec((c, 1), lambda i: (0, 0)),
        ],
        out_specs=pl.BlockSpec((1, c, h * w), lambda i: (i, 0, 0)),
        out_shape=jax.ShapeDtypeStruct((b, c, h * w), jnp.float32),
        compiler_params=pltpu.CompilerParams(
            dimension_semantics=("arbitrary",)),
    )(feat3, mask.reshape(b, 1, h * w), W_sp, W_out, W_text,
      text_emb.reshape(TEXT_DIM, 1), b_sp.reshape(EMBED_DIM, 1),
      b_text.reshape(EMBED_DIM, 1), b_out.reshape(c, 1))

    return out.reshape(b, c, h, w)
